# Initial kernel scaffold; baseline (speedup 1.0000x reference)
import jax
import jax.numpy as jnp
from jax.experimental import pallas as pl


def kernel(x, edge_index, batch, W1, b1, g1, be1, W2, b2, g2, be2, W3, b3, g3, be3):
    raise NotImplementedError("write your pallas kernel here")


# speedup vs baseline: 17.4982x; 17.4982x over previous
import functools

import jax
import jax.numpy as jnp
from jax import lax
from jax.experimental import pallas as pl
from jax.experimental.pallas import tpu as pltpu
from jax.experimental.pallas import tpu_sc as plsc

N = 50000
F = 9
H = 64
HH = 32
E = 800000
B = 64
EPS = 1e-5

NC = 2
NS = 16
NW = NC * NS

IB = 128
RING = 4
G = 8
SC_BLKS = 400
NG = SC_BLKS // G
E_PAD_S = NS * SC_BLKS * IB

DG_BLKS = 196
DG_K = 14
E_PAD_D = NW * DG_BLKS * IB

TRASH = N
NACC = 51200
ZCH = 128
ROWS_PT = NACC // NS
NFL = 50048
FL_PT = NFL // NS

PW = 1568
PW_LAST = N - (NW - 1) * PW

BN_BLK = 2000
N_BLKS = N // BN_BLK

_sc_mesh = plsc.VectorSubcoreMesh(
    core_axis_name="c", subcore_axis_name="s", num_cores=NC, num_subcores=NS)
_sc_params = pltpu.CompilerParams(use_tc_tiling_on_sc=False)


def _sc_deg_body(dstd_hbm, out_hbm, acc, idx_v, ones_b, zb):
  c = lax.axis_index("c")
  s = lax.axis_index("s")
  w = s * NC + c
  pltpu.sync_copy(dstd_hbm.at[w], idx_v)

  one16 = jnp.ones((16,), jnp.float32)
  zero16 = jnp.zeros((16,), jnp.float32)

  def fill(r, _):
    ones_b[r, pl.ds(0, 16)] = one16
    zb[r, pl.ds(0, 16)] = zero16
    return 0
  lax.fori_loop(0, IB, fill, 0)

  def zacc(j, _):
    pltpu.sync_copy(zb, acc.at[pl.ds(s * ROWS_PT + j * ZCH, ZCH)])
    return 0
  lax.fori_loop(0, ROWS_PT // ZCH, zacc, 0)
  plsc.subcore_barrier()

  def batch_fn(bi, sem):
    for j in range(DG_K):
      pltpu.async_copy(ones_b, acc.at[idx_v.at[bi * DG_K + j]], sem, add=True)
    for j in range(DG_K):
      pltpu.make_async_copy(ones_b, acc.at[idx_v.at[bi * DG_K + j]], sem).wait()

  def scatter_all(sem):
    lax.fori_loop(0, DG_BLKS // DG_K,
                  lambda bi, _: (batch_fn(bi, sem), 0)[1], 0)
  pl.run_scoped(scatter_all, pltpu.SemaphoreType.DMA)
  plsc.subcore_barrier()

  def flush(j, _):
    r = s * ROWS_PT + j * ZCH
    pltpu.sync_copy(acc.at[pl.ds(r, ZCH)], ones_b)
    pltpu.sync_copy(ones_b, out_hbm.at[c].at[pl.ds(r, ZCH)])
    return 0
  lax.fori_loop(0, ROWS_PT // ZCH, flush, 0)


def _deg_partials(dstd):
  return pl.kernel(
      _sc_deg_body,
      out_type=jax.ShapeDtypeStruct((NC, NACC, 16), jnp.float32),
      mesh=_sc_mesh,
      compiler_params=_sc_params,
      scratch_types=[
          pltpu.VMEM_SHARED((NACC, 16), jnp.float32),
          pltpu.VMEM((DG_BLKS, IB), jnp.int32),
          pltpu.VMEM((IB, 16), jnp.float32),
          pltpu.VMEM((IB, 16), jnp.float32),
      ],
  )(dstd)


def _sc_gcn_body(gg_hbm, cidx_hbm, out_hbm, acc,
                 cbuf, gb0, gb1, gb2, gb3, zb, gsem, isem):
  c = lax.axis_index("c")
  s = lax.axis_index("s")
  gbufs = (gb0, gb1, gb2, gb3)
  my_idx = cidx_hbm.at[c].at[s]

  zero16 = jnp.zeros((16,), jnp.float32)

  def zrow(r, _):
    zb[r, pl.ds(0, 16)] = zero16
    zb[r, pl.ds(16, 16)] = zero16
    return 0
  lax.fori_loop(0, ZCH, zrow, 0)

  def zacc(j, _):
    pltpu.sync_copy(zb, acc.at[pl.ds(s * ROWS_PT + j * ZCH, ZCH)])
    return 0
  lax.fori_loop(0, ROWS_PT // ZCH, zacc, 0)
  plsc.subcore_barrier()

  pltpu.sync_copy(my_idx.at[pl.ds(0, G)], cbuf.at[0])
  for j in range(RING):
    pltpu.async_copy(gg_hbm.at[cbuf.at[0].at[j].at[0]], gbufs[j], gsem)

  def group(gi, _):
    p = lax.rem(gi, 2)
    pn = lax.rem(gi + 1, 2)

    @pl.when(gi + 1 < NG)
    def _():
      pltpu.async_copy(my_idx.at[pl.ds((gi + 1) * G, G)], cbuf.at[pn], isem)

    for j in range(G):
      slot = gbufs[j % RING]
      pltpu.make_async_copy(
          gg_hbm.at[cbuf.at[p].at[j].at[0]], slot, gsem).wait()
      pltpu.sync_copy(slot, acc.at[cbuf.at[p].at[j].at[1]], add=True)
      if j == G - RING:
        @pl.when(gi + 1 < NG)
        def _():
          pltpu.make_async_copy(
              my_idx.at[pl.ds((gi + 1) * G, G)], cbuf.at[pn], isem).wait()
      if j < G - RING:
        pltpu.async_copy(
            gg_hbm.at[cbuf.at[p].at[j + RING].at[0]], slot, gsem)
      else:
        @pl.when(gi + 1 < NG)
        def _():
          pltpu.async_copy(
              gg_hbm.at[cbuf.at[pn].at[j + RING - G].at[0]], slot, gsem)
    return 0
  lax.fori_loop(0, NG, group, 0)
  plsc.subcore_barrier()

  r = s * FL_PT
  pltpu.sync_copy(acc.at[pl.ds(r, FL_PT)], out_hbm.at[c].at[pl.ds(r, FL_PT)])


def _gcn_scatter(gg_flat, cidx):
  return pl.kernel(
      _sc_gcn_body,
      out_type=jax.ShapeDtypeStruct((NC, NFL, HH), jnp.float32),
      mesh=_sc_mesh,
      compiler_params=_sc_params,
      scratch_types=[
          pltpu.VMEM_SHARED((NACC, HH), jnp.float32),
          pltpu.VMEM((2, G, 2, IB), jnp.int32),
          pltpu.VMEM((IB, HH), jnp.float32),
          pltpu.VMEM((IB, HH), jnp.float32),
          pltpu.VMEM((IB, HH), jnp.float32),
          pltpu.VMEM((IB, HH), jnp.float32),
          pltpu.VMEM((ZCH, HH), jnp.float32),
          pltpu.SemaphoreType.DMA,
          pltpu.SemaphoreType.DMA,
      ],
  )(gg_flat, cidx)


def _sc_pool_body(h_hbm, bt_hbm, out_hbm, hbuf, bbuf, sums, maxs):
  c = lax.axis_index("c")
  s = lax.axis_index("s")
  w = s * NC + c
  start = w * PW

  @pl.when(w < NW - 1)
  def _():
    pltpu.sync_copy(h_hbm.at[pl.ds(start, PW)], hbuf)
    pltpu.sync_copy(bt_hbm.at[pl.ds(start, PW)], bbuf.at[pl.ds(0, PW)])

  @pl.when(w == NW - 1)
  def _():
    pltpu.sync_copy(h_hbm.at[pl.ds(start, PW_LAST)], hbuf.at[pl.ds(0, PW_LAST)])
    pltpu.sync_copy(bt_hbm.at[pl.ds(start, PW_LAST)],
                    bbuf.at[pl.ds(0, PW_LAST)])

  zero16 = jnp.zeros((16,), jnp.float32)
  ninf16 = jnp.full((16,), -jnp.inf, jnp.float32)

  def init(r, _):
    for k in range(H // 16):
      sums[r, pl.ds(16 * k, 16)] = zero16
      maxs[r, pl.ds(16 * k, 16)] = ninf16
    return 0
  lax.fori_loop(0, B, init, 0)

  nr = jnp.where(w == NW - 1, PW_LAST, PW)

  def row(r, _):
    b = bbuf[pl.ds(r, 16)][0]
    for k in range(H // 16):
      hv = hbuf[r, pl.ds(16 * k, 16)]
      sums[b, pl.ds(16 * k, 16)] = sums[b, pl.ds(16 * k, 16)] + hv
      maxs[b, pl.ds(16 * k, 16)] = jnp.maximum(maxs[b, pl.ds(16 * k, 16)], hv)
    return 0
  lax.fori_loop(0, nr, row, 0)

  pltpu.sync_copy(sums, out_hbm.at[w].at[0])
  pltpu.sync_copy(maxs, out_hbm.at[w].at[1])


def _pool_partials(h3, batch):
  return pl.kernel(
      _sc_pool_body,
      out_type=jax.ShapeDtypeStruct((NW, 2, B, H), jnp.float32),
      mesh=_sc_mesh,
      compiler_params=_sc_params,
      scratch_types=[
          pltpu.VMEM((PW, H), jnp.float32),
          pltpu.VMEM((PW + 16,), jnp.int32),
          pltpu.VMEM((B, H), jnp.float32),
          pltpu.VMEM((B, H), jnp.float32),
      ],
  )(h3, batch)


def _tc_dinv_body(dp_ref, dinv_ref):
  d = dp_ref[0, :, 0:1] + dp_ref[1, :, 0:1] + 1.0
  dinv_ref[...] = lax.rsqrt(d)


def _dinv(degp):
  return pl.pallas_call(
      _tc_dinv_body,
      grid=(N_BLKS,),
      in_specs=[pl.BlockSpec((2, BN_BLK, 16), lambda i: (0, i, 0))],
      out_specs=pl.BlockSpec((BN_BLK, 1), lambda i: (i, 0)),
      out_shape=jax.ShapeDtypeStruct((N, 1), jnp.float32),
  )(degp)


def _tc_g1_body(x_ref, w_ref, dinv_ref, g_ref):
  u = jnp.dot(x_ref[...], w_ref[...], preferred_element_type=jnp.float32)
  g = u * dinv_ref[...]
  g_ref[0] = g[:, :HH]
  g_ref[1] = g[:, HH:]


def _g_first(x, W1, dinv):
  return pl.pallas_call(
      _tc_g1_body,
      grid=(N_BLKS,),
      in_specs=[
          pl.BlockSpec((BN_BLK, F), lambda i: (i, 0)),
          pl.BlockSpec((F, H), lambda i: (0, 0)),
          pl.BlockSpec((BN_BLK, 1), lambda i: (i, 0)),
      ],
      out_specs=pl.BlockSpec((2, BN_BLK, HH), lambda i: (0, i, 0)),
      out_shape=jax.ShapeDtypeStruct((2, N, HH), jnp.float32),
  )(x, W1, dinv)


def _tc_z_body(s_ref, g_ref, dinv_ref, b_ref, z_ref, st_ref, acc):
  i = pl.program_id(0)
  z = jnp.concatenate(
      [s_ref[0] + g_ref[0], s_ref[1] + g_ref[1]], axis=1)
  z = z * dinv_ref[...] + b_ref[...]
  z_ref[...] = z
  blk = jnp.concatenate(
      [jnp.sum(z, axis=0, keepdims=True),
       jnp.sum(z * z, axis=0, keepdims=True)], axis=0)

  @pl.when(i == 0)
  def _():
    acc[...] = blk

  @pl.when(i > 0)
  def _():
    acc[...] = acc[...] + blk
  st_ref[...] = acc[...]


def _z_stats(s_, gg, dinv, bias):
  return pl.pallas_call(
      _tc_z_body,
      grid=(N_BLKS,),
      in_specs=[
          pl.BlockSpec((2, BN_BLK, HH), lambda i: (0, i, 0)),
          pl.BlockSpec((2, BN_BLK, HH), lambda i: (0, i, 0)),
          pl.BlockSpec((BN_BLK, 1), lambda i: (i, 0)),
          pl.BlockSpec((1, H), lambda i: (0, 0)),
      ],
      out_specs=[
          pl.BlockSpec((BN_BLK, H), lambda i: (i, 0)),
          pl.BlockSpec((2, H), lambda i: (0, 0)),
      ],
      out_shape=[
          jax.ShapeDtypeStruct((N, H), jnp.float32),
          jax.ShapeDtypeStruct((2, H), jnp.float32),
      ],
      scratch_shapes=[pltpu.VMEM((2, H), jnp.float32)],
  )(s_, gg, dinv, bias)


def _tc_zc_body(s_ref, g_ref, dinv_ref, b_ref, bt_ref,
                z_ref, st_ref, cnt_ref, acc, cacc):
  i = pl.program_id(0)
  z = jnp.concatenate(
      [s_ref[0] + g_ref[0], s_ref[1] + g_ref[1]], axis=1)
  z = z * dinv_ref[...] + b_ref[...]
  z_ref[...] = z
  blk = jnp.concatenate(
      [jnp.sum(z, axis=0, keepdims=True),
       jnp.sum(z * z, axis=0, keepdims=True)], axis=0)
  seg = lax.broadcasted_iota(jnp.int32, (B, BN_BLK), 0)
  onehot = (seg == bt_ref[0]).astype(jnp.float32)
  cblk = jnp.sum(onehot, axis=1, keepdims=True)

  @pl.when(i == 0)
  def _():
    acc[...] = blk
    cacc[...] = cblk

  @pl.when(i > 0)
  def _():
    acc[...] = acc[...] + blk
    cacc[...] = cacc[...] + cblk
  st_ref[...] = acc[...]
  cnt_ref[...] = cacc[...]


def _z_stats_counts(s_, gg, dinv, bias, batch_row):
  return pl.pallas_call(
      _tc_zc_body,
      grid=(N_BLKS,),
      in_specs=[
          pl.BlockSpec((2, BN_BLK, HH), lambda i: (0, i, 0)),
          pl.BlockSpec((2, BN_BLK, HH), lambda i: (0, i, 0)),
          pl.BlockSpec((BN_BLK, 1), lambda i: (i, 0)),
          pl.BlockSpec((1, H), lambda i: (0, 0)),
          pl.BlockSpec((1, 1, BN_BLK), lambda i: (i, 0, 0)),
      ],
      out_specs=[
          pl.BlockSpec((BN_BLK, H), lambda i: (i, 0)),
          pl.BlockSpec((2, H), lambda i: (0, 0)),
          pl.BlockSpec((B, 1), lambda i: (0, 0)),
      ],
      out_shape=[
          jax.ShapeDtypeStruct((N, H), jnp.float32),
          jax.ShapeDtypeStruct((2, H), jnp.float32),
          jax.ShapeDtypeStruct((B, 1), jnp.float32),
      ],
      scratch_shapes=[
          pltpu.VMEM((2, H), jnp.float32),
          pltpu.VMEM((B, 1), jnp.float32),
      ],
  )(s_, gg, dinv, bias, batch_row)


def _bn_relu(z_ref, st_ref, gm_ref, bt_ref):
  st = st_ref[...]
  m = st[0:1, :] * (1.0 / N)
  v = st[1:2, :] * (1.0 / N) - m * m
  rstd = lax.rsqrt(v + EPS)
  return jnp.maximum((z_ref[...] - m) * rstd * gm_ref[...] + bt_ref[...], 0.0)


def _tc_h_body(z_ref, st_ref, gm_ref, bt_ref, w_ref, dinv_ref, g_ref):
  h = _bn_relu(z_ref, st_ref, gm_ref, bt_ref)
  u = jnp.dot(h, w_ref[...], preferred_element_type=jnp.float32)
  g = u * dinv_ref[...]
  g_ref[0] = g[:, :HH]
  g_ref[1] = g[:, HH:]


def _g_next(z, st, gamma, beta, Wn, dinv):
  return pl.pallas_call(
      _tc_h_body,
      grid=(N_BLKS,),
      in_specs=[
          pl.BlockSpec((BN_BLK, H), lambda i: (i, 0)),
          pl.BlockSpec((2, H), lambda i: (0, 0)),
          pl.BlockSpec((1, H), lambda i: (0, 0)),
          pl.BlockSpec((1, H), lambda i: (0, 0)),
          pl.BlockSpec((H, H), lambda i: (0, 0)),
          pl.BlockSpec((BN_BLK, 1), lambda i: (i, 0)),
      ],
      out_specs=pl.BlockSpec((2, BN_BLK, HH), lambda i: (0, i, 0)),
      out_shape=jax.ShapeDtypeStruct((2, N, HH), jnp.float32),
  )(z, st, gamma, beta, Wn, dinv)


def _tc_hf_body(z_ref, st_ref, gm_ref, bt_ref, h_ref):
  h_ref[...] = _bn_relu(z_ref, st_ref, gm_ref, bt_ref)


def _h_final(z, st, gamma, beta):
  return pl.pallas_call(
      _tc_hf_body,
      grid=(N_BLKS,),
      in_specs=[
          pl.BlockSpec((BN_BLK, H), lambda i: (i, 0)),
          pl.BlockSpec((2, H), lambda i: (0, 0)),
          pl.BlockSpec((1, H), lambda i: (0, 0)),
          pl.BlockSpec((1, H), lambda i: (0, 0)),
      ],
      out_specs=pl.BlockSpec((BN_BLK, H), lambda i: (i, 0)),
      out_shape=jax.ShapeDtypeStruct((N, H), jnp.float32),
  )(z, st, gamma, beta)


def _tc_comb_body(p_ref, cnt_ref, o_ref):
  sm = p_ref[0, 0]
  mx = p_ref[0, 1]
  for wdx in range(1, NW):
    sm = sm + p_ref[wdx, 0]
    mx = jnp.maximum(mx, p_ref[wdx, 1])
  o_ref[...] = sm / jnp.maximum(cnt_ref[...], 1.0) + mx


def _combine(parts, cnt):
  return pl.pallas_call(
      _tc_comb_body,
      grid=(1,),
      in_specs=[
          pl.BlockSpec((NW, 2, B, H), lambda i: (0, 0, 0, 0)),
          pl.BlockSpec((B, 1), lambda i: (0, 0)),
      ],
      out_specs=pl.BlockSpec((B, H), lambda i: (0, 0)),
      out_shape=jax.ShapeDtypeStruct((B, H), jnp.float32),
  )(parts, cnt)


def kernel(x, edge_index, batch,
           W1, b1, g1, be1, W2, b2, g2, be2, W3, b3, g3, be3):
  src = edge_index[0]
  dst = edge_index[1]

  dstd = jnp.concatenate(
      [dst, jnp.full((E_PAD_D - E,), TRASH, jnp.int32)]
  ).reshape(NW, DG_BLKS, IB)
  srcr = jnp.concatenate(
      [src, jnp.zeros((E_PAD_S - E,), jnp.int32)]).reshape(NS, SC_BLKS, IB)
  dstr = jnp.concatenate(
      [dst, jnp.full((E_PAD_S - E,), TRASH, jnp.int32)]
  ).reshape(NS, SC_BLKS, IB)
  cidx = jnp.stack([
      jnp.stack([srcr, dstr], axis=2),
      jnp.stack([srcr + N, dstr], axis=2),
  ])

  degp = _deg_partials(dstd)
  dinv = _dinv(degp)

  b1r, b2r, b3r = b1.reshape(1, H), b2.reshape(1, H), b3.reshape(1, H)
  g1r, g2r, g3r = g1.reshape(1, H), g2.reshape(1, H), g3.reshape(1, H)
  be1r, be2r, be3r = be1.reshape(1, H), be2.reshape(1, H), be3.reshape(1, H)

  gg = _g_first(x, W1, dinv)
  s_ = _gcn_scatter(gg.reshape(2 * N, HH), cidx)
  z, st = _z_stats(s_, gg, dinv, b1r)
  gg = _g_next(z, st, g1r, be1r, W2, dinv)

  s_ = _gcn_scatter(gg.reshape(2 * N, HH), cidx)
  z, st = _z_stats(s_, gg, dinv, b2r)
  gg = _g_next(z, st, g2r, be2r, W3, dinv)

  s_ = _gcn_scatter(gg.reshape(2 * N, HH), cidx)
  z, st, cnt = _z_stats_counts(s_, gg, dinv, b3r, batch.reshape(N_BLKS, 1, BN_BLK))
  h3 = _h_final(z, st, g3r, be3r)

  parts = _pool_partials(h3, batch)
  return _combine(parts, cnt)
